# Initial kernel scaffold; baseline (speedup 1.0000x reference)
import jax
import jax.numpy as jnp
from jax.experimental import pallas as pl


def kernel(inputs, table):
    raise NotImplementedError("write your pallas kernel here")


# speedup vs baseline: 49.2834x; 49.2834x over previous
import jax
import jax.numpy as jnp
import numpy as np
from jax import lax
from jax.experimental import pallas as pl
from jax.experimental.pallas import tpu as pltpu
from jax.experimental.pallas import tpu_sc as plsc

N_LEVELS = 16
FPL = 2
LOG2_T = 19
T = 1 << LOG2_T
MASK = T - 1
BASE_RES = 16
MAX_RES = 2048
N_POINTS = 1048576
PER_LEVEL_SCALE = float(np.power(MAX_RES / BASE_RES, 1.0 / N_LEVELS))
PRIME_Y_I32 = np.int32(np.uint32(2654435761).view(np.int32))
RES = [int(np.floor(BASE_RES * (PER_LEVEL_SCALE ** l))) for l in range(N_LEVELS)]

NW = 32
P = 128
NPW = N_POINTS // NW
N_CHUNKS = NPW // P
OUT_D = N_LEVELS * FPL


def _body(xs_hbm, ys_hbm, tbl_hbm, out_hbm,
          xs_v, ys_v, fx_v, fy_v, ib, gb, ob, sem):
    wid = lax.axis_index("s") * 2 + lax.axis_index("c")
    lanes0 = lax.iota(jnp.int32, 16)

    def chunk_body(c, _):
        base = wid * NPW + c * P
        pltpu.sync_copy(xs_hbm.at[pl.ds(base, P)], xs_v)
        pltpu.sync_copy(ys_hbm.at[pl.ds(base, P)], ys_v)

        for l in range(N_LEVELS):
            res = float(RES[l])
            off = 2 * l * T

            def idx_body(i, _):
                s = pl.multiple_of(i * 16, 16)
                x = xs_v[pl.ds(s, 16)]
                y = ys_v[pl.ds(s, 16)]
                px = x * res
                py = y * res
                ipx = px.astype(jnp.int32)
                ipy = py.astype(jnp.int32)
                fx_v[pl.ds(s, 16)] = px - ipx.astype(jnp.float32)
                fy_v[pl.ds(s, 16)] = py - ipy.astype(jnp.float32)
                uy0 = ipy * PRIME_Y_I32
                uy1 = (ipy + 1) * PRIME_Y_I32
                ipx1 = ipx + 1
                h = (((ipx ^ uy0) & MASK) * 2 + off,
                     ((ipx ^ uy1) & MASK) * 2 + off,
                     ((ipx1 ^ uy0) & MASK) * 2 + off,
                     ((ipx1 ^ uy1) & MASK) * 2 + off)
                for k in range(4):
                    ib[2 * k][pl.ds(s, 16)] = h[k]
                    ib[2 * k + 1][pl.ds(s, 16)] = h[k] + 1
                return 0

            lax.fori_loop(0, P // 16, idx_body, 0)

            cps = [pltpu.async_copy(tbl_hbm.at[ib[j]], gb[j], sem)
                   for j in range(8)]
            for cp in cps:
                cp.wait()

            def acc_body(i, _):
                s = pl.multiple_of(i * 16, 16)
                lane = lanes0 + i * 16
                fx = fx_v[pl.ds(s, 16)]
                fy = fy_v[pl.ds(s, 16)]
                gx = 1.0 - fx
                gy = 1.0 - fy
                w = (gx * gy, gx * fy, fx * gy, fx * fy)
                acc0 = jnp.zeros((16,), jnp.float32)
                acc1 = jnp.zeros((16,), jnp.float32)
                for k in range(4):
                    f0 = gb[2 * k][pl.ds(s, 16)]
                    f1 = gb[2 * k + 1][pl.ds(s, 16)]
                    acc0 = acc0 + f0 * w[k]
                    acc1 = acc1 + f1 * w[k]
                col = lane * OUT_D + 2 * l
                plsc.store_scatter(ob, [col], acc0)
                plsc.store_scatter(ob, [col + 1], acc1)
                return 0

            lax.fori_loop(0, P // 16, acc_body, 0)

        pltpu.sync_copy(ob, out_hbm.at[pl.ds(base * OUT_D, P * OUT_D)])
        return 0

    lax.fori_loop(0, N_CHUNKS, chunk_body, 0)


@jax.jit
def _run(xs, ys, tbl):
    mesh = plsc.VectorSubcoreMesh(core_axis_name="c", subcore_axis_name="s")
    return pl.kernel(
        _body,
        out_type=jax.ShapeDtypeStruct((N_POINTS * OUT_D,), jnp.float32),
        mesh=mesh,
        compiler_params=pltpu.CompilerParams(
            needs_layout_passes=False, use_tc_tiling_on_sc=False),
        scratch_types=[
            pltpu.VMEM((P,), jnp.float32),
            pltpu.VMEM((P,), jnp.float32),
            pltpu.VMEM((P,), jnp.float32),
            pltpu.VMEM((P,), jnp.float32),
            [pltpu.VMEM((P,), jnp.int32)] * 8,
            [pltpu.VMEM((P,), jnp.float32)] * 8,
            pltpu.VMEM((P * OUT_D,), jnp.float32),
            pltpu.SemaphoreType.DMA,
        ],
    )(xs, ys, tbl)


def kernel(inputs, table):
    out = _run(inputs[:, 0], inputs[:, 1], table.reshape(-1))
    return out.reshape(N_POINTS, OUT_D)
